# baseline (speedup 1.0000x reference)
import functools

import jax
import jax.numpy as jnp
from jax.experimental import pallas as pl
from jax.experimental.pallas import tpu as pltpu

_SAMPLING_EPS = 1e-05
_NEG = -1e30
_VB = 6144
_RB = 8


def _logits_kernel(hid_ref, idx_ref, temp_ref, emb_ref,
                   out_ref, m_ref, s_ref, *, vocab, vb):
    i = pl.program_id(0)
    idx = idx_ref[...]
    rows = jax.lax.broadcasted_iota(jnp.int32, (1, hid_ref.shape[0]), 1)
    onehot = (idx == rows).astype(jnp.float32)
    pruned = jax.lax.dot_general(
        onehot, hid_ref[...], (((1,), (0,)), ((), ())),
        preferred_element_type=jnp.float32,
        precision=jax.lax.Precision.HIGHEST)
    t = temp_ref[...]
    t = jnp.where(t < _SAMPLING_EPS, 1.0, t)
    raw = jax.lax.dot_general(
        pruned, emb_ref[...], (((1,), (1,)), ((), ())),
        preferred_element_type=jnp.float32,
        precision=jax.lax.Precision.DEFAULT)
    logits = raw / t
    col = i * vb + jax.lax.broadcasted_iota(jnp.int32, logits.shape, 1)
    valid = col < vocab
    out_ref[...] = jnp.where(valid, logits, _NEG)
    lm = jnp.where(valid, logits, -jnp.inf)
    bm = jnp.max(lm, axis=-1, keepdims=True)

    @pl.when(i == 0)
    def _():
        m_ref[...] = bm
        s_ref[...] = jnp.sum(jnp.exp(lm - bm), axis=-1, keepdims=True)

    @pl.when(i > 0)
    def _():
        m_old = m_ref[...]
        m_new = jnp.maximum(m_old, bm)
        s_ref[...] = (s_ref[...] * jnp.exp(m_old - m_new)
                      + jnp.sum(jnp.exp(lm - m_new), axis=-1, keepdims=True))
        m_ref[...] = m_new


def _avg(lo, hi):
    return (lo & hi) + ((lo ^ hi) >> 1)


def _select_kernel(l_ref, m_ref, s_ref, tp_ref, tk_ref,
                   probs_ref, lp_ref, nt_ref, *, vocab):
    l = l_ref[...]
    m = m_ref[...]
    s = s_ref[...]
    tp = tp_ref[...]
    tk = tk_ref[...]
    e = jnp.exp(l - m)
    tps = tp * s
    bits = jax.lax.bitcast_convert_type(l, jnp.int32)
    k = jnp.where(bits >= 0, bits, bits ^ jnp.int32(0x7FFFFFFF))
    idx = jax.lax.broadcasted_iota(jnp.int32, l.shape, 1)
    kmax = jnp.max(k, axis=-1, keepdims=True)
    lo0 = jnp.full_like(kmax, jnp.iinfo(jnp.int32).min)
    hi0 = kmax

    tkm1f = tk.astype(jnp.float32) - 1.0

    def bc_body(_, carry):
        lo, hi = carry
        mid = _avg(lo, hi)
        gt_m = k > mid
        C = jnp.sum(jnp.where(gt_m, 1.0, 0.0), axis=-1, keepdims=True)
        P = jnp.sum(jnp.where(gt_m, e, 0.0), axis=-1, keepdims=True)
        ok = (C <= tkm1f) & (P <= tps)
        return jnp.where(ok, lo, mid), jnp.where(ok, mid, hi)

    tau = jax.lax.fori_loop(0, 32, bc_body, (lo0, hi0))[1]

    gt = k > tau
    eq = k == tau
    Kst = jnp.sum(jnp.where(gt, 1, 0), axis=-1, keepdims=True)
    Pst = jnp.sum(jnp.where(gt, e, 0.0), axis=-1, keepdims=True)
    cst = jnp.sum(jnp.where(eq, 1, 0), axis=-1, keepdims=True)
    pv = jnp.max(jnp.where(eq, e, 0.0), axis=-1, keepdims=True)
    nf = jnp.floor((tps - Pst) / jnp.where(pv > 0, pv, 1.0)) + 1.0
    nf = jnp.clip(nf, 0.0, cst.astype(jnp.float32))
    n_in = jnp.where(pv > 0, nf.astype(jnp.int32), cst)
    mstar = jnp.maximum(jnp.minimum(tk - Kst, n_in), 1)

    def b3_run(_):
        def cond(carry):
            lo, hi = carry
            return jnp.any(lo + 1 < hi)

        def body(carry):
            lo, hi = carry
            mid = (lo + hi) >> 1
            C = jnp.sum(jnp.where(eq & (idx < mid), 1, 0),
                        axis=-1, keepdims=True)
            ok = C >= mstar
            return jnp.where(ok, lo, mid), jnp.where(ok, mid, hi)

        return jax.lax.while_loop(cond, body,
                                  (jnp.zeros_like(kmax),
                                   jnp.full_like(kmax, vocab)))[1]

    bstar = jax.lax.cond(jnp.any(mstar < cst), b3_run,
                         lambda _: jnp.full_like(kmax, vocab), None)

    keep = gt | (eq & (idx < bstar))
    lam = jnp.where(keep, l, _NEG)
    Zk = jnp.sum(jnp.where(keep, e, 0.0), axis=-1, keepdims=True)
    probs_ref[...] = jnp.where(keep, e / Zk, 0.0)
    lp_ref[...] = lam - (m + jnp.log(Zk))
    nt_ref[...] = jnp.min(jnp.where(k == kmax, idx, jnp.int32(0x7FFFFFFF)),
                          axis=-1, keepdims=True)


def kernel(hidden_states, embedding, last_token_indices,
           temperatures, top_ps, top_ks, interpret=False):
    total, d = hidden_states.shape
    vocab = embedding.shape[0]
    b = last_token_indices.shape[0]
    idx2 = last_token_indices.astype(jnp.int32).reshape(b, 1)
    temp2 = temperatures.reshape(b, 1)
    tp2 = top_ps.reshape(b, 1)
    tk2 = top_ks.astype(jnp.int32).reshape(b, 1)

    nv = pl.cdiv(vocab, _VB)
    vp = nv * _VB
    logits, m, s = pl.pallas_call(
        functools.partial(_logits_kernel, vocab=vocab, vb=_VB),
        grid=(nv,),
        in_specs=[
            pl.BlockSpec((total, d), lambda i: (0, 0)),
            pl.BlockSpec((b, 1), lambda i: (0, 0)),
            pl.BlockSpec((b, 1), lambda i: (0, 0)),
            pl.BlockSpec((_VB, d), lambda i: (i, 0)),
        ],
        out_specs=[
            pl.BlockSpec((b, _VB), lambda i: (0, i)),
            pl.BlockSpec((b, 1), lambda i: (0, 0)),
            pl.BlockSpec((b, 1), lambda i: (0, 0)),
        ],
        out_shape=[
            jax.ShapeDtypeStruct((b, vp), jnp.float32),
            jax.ShapeDtypeStruct((b, 1), jnp.float32),
            jax.ShapeDtypeStruct((b, 1), jnp.float32),
        ],
        compiler_params=pltpu.CompilerParams(
            dimension_semantics=("arbitrary",)),
        interpret=interpret,
    )(hidden_states, idx2, temp2, embedding)

    nr = b // _RB
    probs, lp, nt = pl.pallas_call(
        functools.partial(_select_kernel, vocab=vocab),
        grid=(nr,),
        in_specs=[
            pl.BlockSpec((_RB, vp), lambda r: (r, 0)),
            pl.BlockSpec((_RB, 1), lambda r: (r, 0)),
            pl.BlockSpec((_RB, 1), lambda r: (r, 0)),
            pl.BlockSpec((_RB, 1), lambda r: (r, 0)),
            pl.BlockSpec((_RB, 1), lambda r: (r, 0)),
        ],
        out_specs=[
            pl.BlockSpec((_RB, vp), lambda r: (r, 0)),
            pl.BlockSpec((_RB, vp), lambda r: (r, 0)),
            pl.BlockSpec((_RB, 1), lambda r: (r, 0)),
        ],
        out_shape=[
            jax.ShapeDtypeStruct((b, vp), jnp.float32),
            jax.ShapeDtypeStruct((b, vp), jnp.float32),
            jax.ShapeDtypeStruct((b, 1), jnp.int32),
        ],
        compiler_params=pltpu.CompilerParams(
            dimension_semantics=("arbitrary",)),
        interpret=interpret,
    )(logits, m, s, tp2, tk2)

    return probs[:, :vocab], lp[:, :vocab], nt.reshape(b)


# speedup vs baseline: 1.0186x; 1.0186x over previous
import functools

import jax
import jax.numpy as jnp
from jax.experimental import pallas as pl
from jax.experimental.pallas import tpu as pltpu

_SAMPLING_EPS = 1e-05
_NEG = -1e30
_VB = 4096
_RB = 8


def _logits_kernel(hid_ref, idx_ref, temp_ref, emb_ref,
                   out_ref, m_ref, s_ref, *, vocab, vb):
    i = pl.program_id(0)
    idx = idx_ref[...]
    rows = jax.lax.broadcasted_iota(jnp.int32, (1, hid_ref.shape[0]), 1)
    onehot = (idx == rows).astype(jnp.float32)
    pruned = jax.lax.dot_general(
        onehot, hid_ref[...], (((1,), (0,)), ((), ())),
        preferred_element_type=jnp.float32,
        precision=jax.lax.Precision.HIGHEST)
    t = temp_ref[...]
    t = jnp.where(t < _SAMPLING_EPS, 1.0, t)
    raw = jax.lax.dot_general(
        pruned, emb_ref[...], (((1,), (1,)), ((), ())),
        preferred_element_type=jnp.float32,
        precision=jax.lax.Precision.DEFAULT)
    logits = raw / t
    col = i * vb + jax.lax.broadcasted_iota(jnp.int32, logits.shape, 1)
    valid = col < vocab
    out_ref[...] = jnp.where(valid, logits, _NEG)
    lm = jnp.where(valid, logits, -jnp.inf)
    bm = jnp.max(lm, axis=-1, keepdims=True)

    @pl.when(i == 0)
    def _():
        m_ref[...] = bm
        s_ref[...] = jnp.sum(jnp.exp(lm - bm), axis=-1, keepdims=True)

    @pl.when(i > 0)
    def _():
        m_old = m_ref[...]
        m_new = jnp.maximum(m_old, bm)
        s_ref[...] = (s_ref[...] * jnp.exp(m_old - m_new)
                      + jnp.sum(jnp.exp(lm - m_new), axis=-1, keepdims=True))
        m_ref[...] = m_new


def _avg(lo, hi):
    return (lo & hi) + ((lo ^ hi) >> 1)


def _select_kernel(l_ref, m_ref, s_ref, tp_ref, tk_ref,
                   probs_ref, lp_ref, nt_ref, *, vocab):
    l = l_ref[...]
    m = m_ref[...]
    s = s_ref[...]
    tp = tp_ref[...]
    tk = tk_ref[...]
    e = jnp.exp(l - m)
    tps = tp * s
    bits = jax.lax.bitcast_convert_type(l, jnp.int32)
    k = jnp.where(bits >= 0, bits, bits ^ jnp.int32(0x7FFFFFFF))
    idx = jax.lax.broadcasted_iota(jnp.int32, l.shape, 1)
    kmax = jnp.max(k, axis=-1, keepdims=True)
    lo0 = jnp.full_like(kmax, jnp.iinfo(jnp.int32).min)
    hi0 = kmax

    tkm1f = tk.astype(jnp.float32) - 1.0

    def bc_body(_, carry):
        lo, hi = carry
        mid = _avg(lo, hi)
        gt_m = k > mid
        C = jnp.sum(jnp.where(gt_m, 1.0, 0.0), axis=-1, keepdims=True)
        P = jnp.sum(jnp.where(gt_m, e, 0.0), axis=-1, keepdims=True)
        ok = (C <= tkm1f) & (P <= tps)
        return jnp.where(ok, lo, mid), jnp.where(ok, mid, hi)

    tau = jax.lax.fori_loop(0, 32, bc_body, (lo0, hi0))[1]

    gt = k > tau
    eq = k == tau
    Kst = jnp.sum(jnp.where(gt, 1, 0), axis=-1, keepdims=True)
    Pst = jnp.sum(jnp.where(gt, e, 0.0), axis=-1, keepdims=True)
    cst = jnp.sum(jnp.where(eq, 1, 0), axis=-1, keepdims=True)
    pv = jnp.max(jnp.where(eq, e, 0.0), axis=-1, keepdims=True)
    nf = jnp.floor((tps - Pst) / jnp.where(pv > 0, pv, 1.0)) + 1.0
    nf = jnp.clip(nf, 0.0, cst.astype(jnp.float32))
    n_in = jnp.where(pv > 0, nf.astype(jnp.int32), cst)
    mstar = jnp.maximum(jnp.minimum(tk - Kst, n_in), 1)

    def b3_run(_):
        def cond(carry):
            lo, hi = carry
            return jnp.any(lo + 1 < hi)

        def body(carry):
            lo, hi = carry
            mid = (lo + hi) >> 1
            C = jnp.sum(jnp.where(eq & (idx < mid), 1, 0),
                        axis=-1, keepdims=True)
            ok = C >= mstar
            return jnp.where(ok, lo, mid), jnp.where(ok, mid, hi)

        return jax.lax.while_loop(cond, body,
                                  (jnp.zeros_like(kmax),
                                   jnp.full_like(kmax, vocab)))[1]

    bstar = jax.lax.cond(jnp.any(mstar < cst), b3_run,
                         lambda _: jnp.full_like(kmax, vocab), None)

    keep = gt | (eq & (idx < bstar))
    lam = jnp.where(keep, l, _NEG)
    Zk = jnp.sum(jnp.where(keep, e, 0.0), axis=-1, keepdims=True)
    probs_ref[...] = jnp.where(keep, e / Zk, 0.0)
    lp_ref[...] = lam - (m + jnp.log(Zk))
    nt_ref[...] = jnp.min(jnp.where(k == kmax, idx, jnp.int32(0x7FFFFFFF)),
                          axis=-1, keepdims=True)


def kernel(hidden_states, embedding, last_token_indices,
           temperatures, top_ps, top_ks, interpret=False):
    total, d = hidden_states.shape
    vocab = embedding.shape[0]
    b = last_token_indices.shape[0]
    idx2 = last_token_indices.astype(jnp.int32).reshape(b, 1)
    temp2 = temperatures.reshape(b, 1)
    tp2 = top_ps.reshape(b, 1)
    tk2 = top_ks.astype(jnp.int32).reshape(b, 1)

    nv = pl.cdiv(vocab, _VB)
    vp = nv * _VB
    logits, m, s = pl.pallas_call(
        functools.partial(_logits_kernel, vocab=vocab, vb=_VB),
        grid=(nv,),
        in_specs=[
            pl.BlockSpec((total, d), lambda i: (0, 0)),
            pl.BlockSpec((b, 1), lambda i: (0, 0)),
            pl.BlockSpec((b, 1), lambda i: (0, 0)),
            pl.BlockSpec((_VB, d), lambda i: (i, 0)),
        ],
        out_specs=[
            pl.BlockSpec((b, _VB), lambda i: (0, i)),
            pl.BlockSpec((b, 1), lambda i: (0, 0)),
            pl.BlockSpec((b, 1), lambda i: (0, 0)),
        ],
        out_shape=[
            jax.ShapeDtypeStruct((b, vp), jnp.float32),
            jax.ShapeDtypeStruct((b, 1), jnp.float32),
            jax.ShapeDtypeStruct((b, 1), jnp.float32),
        ],
        compiler_params=pltpu.CompilerParams(
            dimension_semantics=("arbitrary",)),
        interpret=interpret,
    )(hidden_states, idx2, temp2, embedding)

    nr = b // _RB
    probs, lp, nt = pl.pallas_call(
        functools.partial(_select_kernel, vocab=vocab),
        grid=(nr,),
        in_specs=[
            pl.BlockSpec((_RB, vp), lambda r: (r, 0)),
            pl.BlockSpec((_RB, 1), lambda r: (r, 0)),
            pl.BlockSpec((_RB, 1), lambda r: (r, 0)),
            pl.BlockSpec((_RB, 1), lambda r: (r, 0)),
            pl.BlockSpec((_RB, 1), lambda r: (r, 0)),
        ],
        out_specs=[
            pl.BlockSpec((_RB, vp), lambda r: (r, 0)),
            pl.BlockSpec((_RB, vp), lambda r: (r, 0)),
            pl.BlockSpec((_RB, 1), lambda r: (r, 0)),
        ],
        out_shape=[
            jax.ShapeDtypeStruct((b, vp), jnp.float32),
            jax.ShapeDtypeStruct((b, vp), jnp.float32),
            jax.ShapeDtypeStruct((b, 1), jnp.int32),
        ],
        compiler_params=pltpu.CompilerParams(
            dimension_semantics=("arbitrary",)),
        interpret=interpret,
    )(logits, m, s, tp2, tk2)

    return probs[:, :vocab], lp[:, :vocab], nt.reshape(b)
